# baseline (speedup 1.0000x reference)
import functools

import jax
import jax.numpy as jnp
from jax import lax
from jax.experimental import pallas as pl
from jax.experimental.pallas import tpu as pltpu
from jax.experimental.pallas import tpu_sc as plsc

B = 128
N = 100000
BR = 8
GRID = B // BR
W = 1024
NCHUNK = 97
TAIL = N - NCHUNK * W

_KS0 = 0
_KS1 = 42
_KS2 = _KS0 ^ _KS1 ^ 0x1BD11BDA

_ROT_A = (13, 15, 26, 6)
_ROT_B = (17, 29, 16, 24)

_LN2 = 0.6931471805599453


def _threefry_bits(idx):
    ks = (jnp.uint32(_KS0), jnp.uint32(_KS1), jnp.uint32(_KS2))
    x0 = jnp.full_like(idx, ks[0])
    x1 = idx + ks[1]
    for g in range(5):
        rots = _ROT_A if g % 2 == 0 else _ROT_B
        for r in rots:
            x0 = x0 + x1
            x1 = (x1 << r) | (x1 >> (32 - r))
            x1 = x1 ^ x0
        x0 = x0 + ks[(g + 1) % 3]
        x1 = x1 + ks[(g + 2) % 3] + jnp.uint32(g + 1)
    return x0 ^ x1


def _t2(idx):
    bits = _threefry_bits(idx)
    fb = (bits >> 9) | jnp.uint32(0x3F800000)
    u = lax.bitcast_convert_type(fb, jnp.float32) - jnp.float32(1.0)
    tiny = jnp.float32(jnp.finfo(jnp.float32).tiny)
    u = jnp.maximum(tiny, u * (jnp.float32(1.0) - tiny) + tiny)
    return -jnp.log2(u)


def _sample_body(x_ref, y_ref):
    pid = pl.program_id(0)
    row_u = lax.broadcasted_iota(jnp.uint32, (BR, W), 0)
    col_u = lax.broadcasted_iota(jnp.uint32, (BR, W), 1)
    base = (jnp.uint32(pid) * jnp.uint32(BR) + row_u) * jnp.uint32(N) + col_u
    lane_i = lax.broadcasted_iota(jnp.int32, (BR, W), 1)
    ln2 = jnp.float32(_LN2)

    neg_inf = jnp.float32(-jnp.inf)
    vmax0 = jnp.full((BR, W), neg_inf, jnp.float32)
    vidx0 = jnp.full((BR, W), 0x7FFFFFFF, jnp.int32)
    UNROLL = 2

    def update(c, carry, xc, t2):
        vmax, vidx = carry
        val = xc - ln2 * jnp.log2(t2)
        upd = val > vmax
        vmax_new = jnp.where(upd, val, vmax)
        vidx_new = jnp.where(upd, c * W + lane_i, vidx)
        return vmax_new, vidx_new

    def stepu(i, carry):
        c0 = i * UNROLL
        xs = [x_ref[:, pl.ds((c0 + k) * W, W)] for k in range(UNROLL)]
        ts = [_t2(base + jnp.uint32((c0 + k) * W)) for k in range(UNROLL)]
        for k in range(UNROLL):
            carry = update(c0 + k, carry, xs[k], ts[k])
        return carry

    carry = lax.fori_loop(0, NCHUNK // UNROLL, stepu, (vmax0, vidx0))
    for cL in range((NCHUNK // UNROLL) * UNROLL, NCHUNK):
        xL = x_ref[:, pl.ds(cL * W, W)]
        carry = update(cL, carry, xL, _t2(base + jnp.uint32(cL * W)))
    vmax, vidx = carry

    xt = x_ref[:, NCHUNK * W:N]
    row_t = lax.broadcasted_iota(jnp.uint32, (BR, TAIL), 0)
    col_t = lax.broadcasted_iota(jnp.uint32, (BR, TAIL), 1)
    idx_t = ((jnp.uint32(pid) * jnp.uint32(BR) + row_t) * jnp.uint32(N)
             + jnp.uint32(NCHUNK * W) + col_t)
    val_t = xt - ln2 * jnp.log2(_t2(idx_t))
    t_vmax = jnp.max(val_t, axis=1, keepdims=True)
    lane_t = lax.broadcasted_iota(jnp.int32, (BR, TAIL), 1)
    big = jnp.int32(0x7FFFFFFF)
    t_vidx = jnp.min(jnp.where(val_t == t_vmax, NCHUNK * W + lane_t, big),
                     axis=1, keepdims=True)

    gmax = jnp.maximum(jnp.max(vmax, axis=1, keepdims=True), t_vmax)
    cand = jnp.min(jnp.where(vmax == gmax, vidx, big), axis=1, keepdims=True)
    cand_t = jnp.where(t_vmax == gmax, t_vidx, big)
    y = jnp.minimum(cand, cand_t)
    y_ref[0, 0, :] = y[:, 0]


_SC_INFO = plsc.get_sparse_core_info()
_NC = _SC_INFO.num_cores
_NS = _SC_INFO.num_subcores
_NW = _NC * _NS
_ROWS_PER_W = B // _NW
_NV = N // 16
_SU = 10

_sc_mesh = plsc.VectorSubcoreMesh(core_axis_name="c", subcore_axis_name="s")


@functools.partial(
    pl.kernel,
    mesh=_sc_mesh,
    out_type=jax.ShapeDtypeStruct((B, N), jnp.float32),
    scratch_types=[pltpu.VMEM((N,), jnp.float32),
                   pltpu.VMEM((16,), jnp.float32)],
)
def _sc_softmax(x_hbm, out_hbm, row_v, red_v):
    wid = lax.axis_index("s") * _NC + lax.axis_index("c")

    def do_row(j, _):
        r = wid * _ROWS_PER_W + j
        pltpu.sync_copy(x_hbm.at[r], row_v)

        def maxstep(i, m16):
            b = i * _SU * 16
            for k in range(_SU):
                m16 = jnp.maximum(m16, row_v[pl.ds(b + k * 16, 16)])
            return m16

        m16 = lax.fori_loop(0, _NV // _SU, maxstep,
                            jnp.full((16,), -jnp.inf, jnp.float32))
        M = m16[0]
        for t in range(1, 16):
            M = jnp.maximum(M, m16[t])

        def expstep(i, s16):
            b = i * _SU * 16
            for k in range(_SU):
                sl = pl.ds(b + k * 16, 16)
                e = jnp.exp(row_v[sl] - M)
                row_v[sl] = e
                s16 = s16 + e
            return s16

        s16 = lax.fori_loop(0, _NV // _SU, expstep,
                            jnp.zeros((16,), jnp.float32))
        S = s16[0]
        for t in range(1, 16):
            S = S + s16[t]
        inv = jnp.ones((16,), jnp.float32) / jnp.full((16,), S, jnp.float32)

        def scalestep(i, _):
            b = i * _SU * 16
            for k in range(_SU):
                sl = pl.ds(b + k * 16, 16)
                row_v[sl] = row_v[sl] * inv
            return 0

        lax.fori_loop(0, _NV // _SU, scalestep, 0)
        pltpu.sync_copy(row_v, out_hbm.at[r])
        return 0

    lax.fori_loop(0, _ROWS_PER_W, do_row, 0)


@jax.jit
def kernel(x):
    probs = _sc_softmax(x)
    y3 = pl.pallas_call(
        _sample_body,
        grid=(GRID,),
        in_specs=[pl.BlockSpec((BR, N), lambda i: (i, 0))],
        out_specs=pl.BlockSpec((1, 1, BR), lambda i: (i, 0, 0)),
        out_shape=jax.ShapeDtypeStruct((GRID, 1, BR), jnp.int32),
    )(x)
    return (y3.reshape(B), probs)


# speedup vs baseline: 1.0010x; 1.0010x over previous
import functools

import jax
import jax.numpy as jnp
from jax import lax
from jax.experimental import pallas as pl
from jax.experimental.pallas import tpu as pltpu
from jax.experimental.pallas import tpu_sc as plsc

B = 128
N = 100000
BR = 8
GRID = B // BR
W = 1024
NCHUNK = 97
TAIL = N - NCHUNK * W

_KS0 = 0
_KS1 = 42
_KS2 = _KS0 ^ _KS1 ^ 0x1BD11BDA

_ROT_A = (13, 15, 26, 6)
_ROT_B = (17, 29, 16, 24)

_LN2 = 0.6931471805599453


def _threefry_bits(idx):
    ks = (jnp.uint32(_KS0), jnp.uint32(_KS1), jnp.uint32(_KS2))
    x0 = jnp.full_like(idx, ks[0])
    x1 = idx + ks[1]
    for g in range(5):
        rots = _ROT_A if g % 2 == 0 else _ROT_B
        for r in rots:
            x0 = x0 + x1
            x1 = (x1 << r) | (x1 >> (32 - r))
            x1 = x1 ^ x0
        x0 = x0 + ks[(g + 1) % 3]
        x1 = x1 + ks[(g + 2) % 3] + jnp.uint32(g + 1)
    return x0 ^ x1


def _t2(idx):
    bits = _threefry_bits(idx)
    fb = (bits >> 9) | jnp.uint32(0x3F800000)
    u = lax.bitcast_convert_type(fb, jnp.float32) - jnp.float32(1.0)
    tiny = jnp.float32(jnp.finfo(jnp.float32).tiny)
    u = jnp.maximum(tiny, u * (jnp.float32(1.0) - tiny) + tiny)
    return -jnp.log2(u)


def _sample_body(x_ref, y_ref):
    pid = pl.program_id(0)
    row_u = lax.broadcasted_iota(jnp.uint32, (BR, W), 0)
    col_u = lax.broadcasted_iota(jnp.uint32, (BR, W), 1)
    base = (jnp.uint32(pid) * jnp.uint32(BR) + row_u) * jnp.uint32(N) + col_u
    lane_i = lax.broadcasted_iota(jnp.int32, (BR, W), 1)
    ln2 = jnp.float32(_LN2)

    neg_inf = jnp.float32(-jnp.inf)
    vmax0 = jnp.full((BR, W), neg_inf, jnp.float32)
    vidx0 = jnp.full((BR, W), 0x7FFFFFFF, jnp.int32)
    UNROLL = 3

    def update(c, carry, xc, t2):
        vmax, vidx = carry
        val = xc - ln2 * jnp.log2(t2)
        upd = val > vmax
        vmax_new = jnp.where(upd, val, vmax)
        vidx_new = jnp.where(upd, c * W + lane_i, vidx)
        return vmax_new, vidx_new

    def stepu(i, carry):
        c0 = i * UNROLL
        xs = [x_ref[:, pl.ds((c0 + k) * W, W)] for k in range(UNROLL)]
        ts = [_t2(base + jnp.uint32((c0 + k) * W)) for k in range(UNROLL)]
        for k in range(UNROLL):
            carry = update(c0 + k, carry, xs[k], ts[k])
        return carry

    carry = lax.fori_loop(0, NCHUNK // UNROLL, stepu, (vmax0, vidx0))
    for cL in range((NCHUNK // UNROLL) * UNROLL, NCHUNK):
        xL = x_ref[:, pl.ds(cL * W, W)]
        carry = update(cL, carry, xL, _t2(base + jnp.uint32(cL * W)))
    vmax, vidx = carry

    xt = x_ref[:, NCHUNK * W:N]
    row_t = lax.broadcasted_iota(jnp.uint32, (BR, TAIL), 0)
    col_t = lax.broadcasted_iota(jnp.uint32, (BR, TAIL), 1)
    idx_t = ((jnp.uint32(pid) * jnp.uint32(BR) + row_t) * jnp.uint32(N)
             + jnp.uint32(NCHUNK * W) + col_t)
    val_t = xt - ln2 * jnp.log2(_t2(idx_t))
    t_vmax = jnp.max(val_t, axis=1, keepdims=True)
    lane_t = lax.broadcasted_iota(jnp.int32, (BR, TAIL), 1)
    big = jnp.int32(0x7FFFFFFF)
    t_vidx = jnp.min(jnp.where(val_t == t_vmax, NCHUNK * W + lane_t, big),
                     axis=1, keepdims=True)

    gmax = jnp.maximum(jnp.max(vmax, axis=1, keepdims=True), t_vmax)
    cand = jnp.min(jnp.where(vmax == gmax, vidx, big), axis=1, keepdims=True)
    cand_t = jnp.where(t_vmax == gmax, t_vidx, big)
    y = jnp.minimum(cand, cand_t)
    y_ref[0, 0, :] = y[:, 0]


_SC_INFO = plsc.get_sparse_core_info()
_NC = _SC_INFO.num_cores
_NS = _SC_INFO.num_subcores
_NW = _NC * _NS
_ROWS_PER_W = B // _NW
_NV = N // 16
_SU = 10

_sc_mesh = plsc.VectorSubcoreMesh(core_axis_name="c", subcore_axis_name="s")


@functools.partial(
    pl.kernel,
    mesh=_sc_mesh,
    out_type=jax.ShapeDtypeStruct((B, N), jnp.float32),
    scratch_types=[pltpu.VMEM((N,), jnp.float32),
                   pltpu.VMEM((16,), jnp.float32)],
)
def _sc_softmax(x_hbm, out_hbm, row_v, red_v):
    wid = lax.axis_index("s") * _NC + lax.axis_index("c")

    def do_row(j, _):
        r = wid * _ROWS_PER_W + j
        pltpu.sync_copy(x_hbm.at[r], row_v)

        def maxstep(i, m16):
            b = i * _SU * 16
            for k in range(_SU):
                m16 = jnp.maximum(m16, row_v[pl.ds(b + k * 16, 16)])
            return m16

        m16 = lax.fori_loop(0, _NV // _SU, maxstep,
                            jnp.full((16,), -jnp.inf, jnp.float32))
        M = m16[0]
        for t in range(1, 16):
            M = jnp.maximum(M, m16[t])

        def expstep(i, s16):
            b = i * _SU * 16
            for k in range(_SU):
                sl = pl.ds(b + k * 16, 16)
                e = jnp.exp(row_v[sl] - M)
                row_v[sl] = e
                s16 = s16 + e
            return s16

        s16 = lax.fori_loop(0, _NV // _SU, expstep,
                            jnp.zeros((16,), jnp.float32))
        S = s16[0]
        for t in range(1, 16):
            S = S + s16[t]
        inv = jnp.ones((16,), jnp.float32) / jnp.full((16,), S, jnp.float32)

        def scalestep(i, _):
            b = i * _SU * 16
            for k in range(_SU):
                sl = pl.ds(b + k * 16, 16)
                row_v[sl] = row_v[sl] * inv
            return 0

        lax.fori_loop(0, _NV // _SU, scalestep, 0)
        pltpu.sync_copy(row_v, out_hbm.at[r])
        return 0

    lax.fori_loop(0, _ROWS_PER_W, do_row, 0)


@jax.jit
def kernel(x):
    probs = _sc_softmax(x)
    y3 = pl.pallas_call(
        _sample_body,
        grid=(GRID,),
        in_specs=[pl.BlockSpec((BR, N), lambda i: (i, 0))],
        out_specs=pl.BlockSpec((1, 1, BR), lambda i: (i, 0, 0)),
        out_shape=jax.ShapeDtypeStruct((GRID, 1, BR), jnp.int32),
    )(x)
    return (y3.reshape(B), probs)


# speedup vs baseline: 1.0681x; 1.0670x over previous
import functools

import jax
import jax.numpy as jnp
from jax import lax
from jax.experimental import pallas as pl
from jax.experimental.pallas import tpu as pltpu
from jax.experimental.pallas import tpu_sc as plsc

B = 128
N = 100000
BR = 8
GRID = B // BR
W = 512
NCHUNK = 195
TAIL = N - NCHUNK * W
CS = 128
CLO = CS * W
HIW = N - CLO
NHI = NCHUNK - CS

_KS0 = 0
_KS1 = 42
_KS2 = _KS0 ^ _KS1 ^ 0x1BD11BDA

_ROT_A = (13, 15, 26, 6)
_ROT_B = (17, 29, 16, 24)


def _threefry_bits(idx):
    ks = (jnp.uint32(_KS0), jnp.uint32(_KS1), jnp.uint32(_KS2))
    x0 = jnp.full_like(idx, ks[0])
    x1 = idx + ks[1]
    for g in range(5):
        rots = _ROT_A if g % 2 == 0 else _ROT_B
        for r in rots:
            x0 = x0 + x1
            x1 = (x1 << r) | (x1 >> (32 - r))
            x1 = x1 ^ x0
        x0 = x0 + ks[(g + 1) % 3]
        x1 = x1 + ks[(g + 2) % 3] + jnp.uint32(g + 1)
    return x0 ^ x1


def _t2_from_bits(bits):
    fb = (bits >> 9) | jnp.uint32(0x3F800000)
    u = lax.bitcast_convert_type(fb, jnp.float32) - jnp.float32(1.0)
    tiny = jnp.float32(jnp.finfo(jnp.float32).tiny)
    u = jnp.maximum(tiny, u * (jnp.float32(1.0) - tiny) + tiny)
    return -jnp.log2(u)


def _t2(idx):
    return _t2_from_bits(_threefry_bits(idx))


def _body_a(x_ref, probs_ref, ehi_ref, wmin_ref, widx_ref):
    pid = pl.program_id(0)
    row_u = lax.broadcasted_iota(jnp.uint32, (BR, W), 0)
    col_u = lax.broadcasted_iota(jnp.uint32, (BR, W), 1)
    base = (jnp.uint32(pid) * jnp.uint32(BR) + row_u) * jnp.uint32(N) + col_u
    lane_i = lax.broadcasted_iota(jnp.int32, (BR, W), 1)

    neg_inf = jnp.float32(-jnp.inf)
    MU = 5

    def maxstep(i, m):
        c0 = i * MU
        for k in range(MU):
            m = jnp.maximum(m, x_ref[:, pl.ds((c0 + k) * W, W)])
        return m

    m = lax.fori_loop(0, NCHUNK // MU, maxstep,
                      jnp.full((BR, W), neg_inf, jnp.float32))
    xt = x_ref[:, NCHUNK * W:N]
    t_m = jnp.max(xt, axis=1, keepdims=True)
    M = jnp.maximum(jnp.max(m, axis=1, keepdims=True), t_m)

    s0 = jnp.zeros((BR, W), jnp.float32)
    wmin0 = jnp.full((BR, W), jnp.float32(jnp.inf), jnp.float32)
    widx0 = jnp.full((BR, W), 0x7FFFFFFF, jnp.int32)
    UNROLL = 4

    def update(c, carry, xc, t2):
        s, wmin, widx = carry
        e = jnp.exp(xc - M)
        probs_ref[:, pl.ds(c * W, W)] = e
        s_new = s + e
        w = t2 / e
        upd = w < wmin
        wmin_new = jnp.where(upd, w, wmin)
        widx_new = jnp.where(upd, c * W + lane_i, widx)
        return s_new, wmin_new, widx_new

    def stepu(i, carry):
        c0 = i * UNROLL
        xs = [x_ref[:, pl.ds((c0 + k) * W, W)] for k in range(UNROLL)]
        ts = [_t2(base + jnp.uint32((c0 + k) * W)) for k in range(UNROLL)]
        for k in range(UNROLL):
            carry = update(c0 + k, carry, xs[k], ts[k])
        return carry

    s, wmin, widx = lax.fori_loop(0, CS // UNROLL, stepu,
                                  (s0, wmin0, widx0))
    wmin_ref[...] = wmin[None]
    widx_ref[...] = widx[None]

    def histep(c, s):
        xc = x_ref[:, pl.ds((CS + c) * W, W)]
        e = jnp.exp(xc - M)
        probs_ref[:, pl.ds((CS + c) * W, W)] = e
        ehi_ref[:, pl.ds(c * W, W)] = e
        return s + e

    s = lax.fori_loop(0, NHI, histep, s)
    e_t = jnp.exp(xt - M)
    t_s = jnp.sum(e_t, axis=1, keepdims=True)
    ehi_ref[:, NHI * W:HIW] = e_t
    S = jnp.sum(s, axis=1, keepdims=True) + t_s
    inv_s = jnp.float32(1.0) / S

    def storep(i, _):
        c0 = i * MU
        for k in range(MU):
            sl = pl.ds((c0 + k) * W, W)
            probs_ref[:, sl] = probs_ref[:, sl] * inv_s
        return 0

    lax.fori_loop(0, NCHUNK // MU, storep, 0)
    probs_ref[:, NCHUNK * W:N] = e_t * inv_s


def _body_b(bits_ref, ehi_ref, wmin_ref, widx_ref, y_ref):
    lane_i = lax.broadcasted_iota(jnp.int32, (BR, W), 1)
    wmin0 = jnp.full((BR, W), jnp.float32(jnp.inf), jnp.float32)
    widx0 = jnp.full((BR, W), 0x7FFFFFFF, jnp.int32)
    UNROLL = 4

    def update(c, carry, bc, ec):
        wmin, widx = carry
        w = _t2_from_bits(bc) / ec
        upd = w < wmin
        wmin_new = jnp.where(upd, w, wmin)
        widx_new = jnp.where(upd, CLO + c * W + lane_i, widx)
        return wmin_new, widx_new

    def stepu(i, carry):
        c0 = i * UNROLL
        for k in range(UNROLL):
            sl = pl.ds((c0 + k) * W, W)
            carry = update(c0 + k, carry, bits_ref[:, sl], ehi_ref[:, sl])
        return carry

    carry = lax.fori_loop(0, NHI // UNROLL, stepu, (wmin0, widx0))
    for cL in range((NHI // UNROLL) * UNROLL, NHI):
        sl = pl.ds(cL * W, W)
        carry = update(cL, carry, bits_ref[:, sl], ehi_ref[:, sl])
    wmin_h, widx_h = carry

    b_t = bits_ref[:, NHI * W:HIW]
    e_t = ehi_ref[:, NHI * W:HIW]
    w_t = _t2_from_bits(b_t) / e_t
    t_wmin = jnp.min(w_t, axis=1, keepdims=True)
    lane_t = lax.broadcasted_iota(jnp.int32, (BR, HIW - NHI * W), 1)
    big = jnp.int32(0x7FFFFFFF)
    t_widx = jnp.min(
        jnp.where(w_t == t_wmin, CLO + NHI * W + lane_t, big),
        axis=1, keepdims=True)

    wmin_l = wmin_ref[0]
    widx_l = widx_ref[0]
    gmin = jnp.minimum(
        jnp.minimum(jnp.min(wmin_h, axis=1, keepdims=True), t_wmin),
        jnp.min(wmin_l, axis=1, keepdims=True))
    cand_h = jnp.min(jnp.where(wmin_h == gmin, widx_h, big),
                     axis=1, keepdims=True)
    cand_l = jnp.min(jnp.where(wmin_l == gmin, widx_l, big),
                     axis=1, keepdims=True)
    cand_t = jnp.where(t_wmin == gmin, t_widx, big)
    y = jnp.minimum(jnp.minimum(cand_l, cand_h), cand_t)
    y_ref[0, 0, :] = y[:, 0]


_SC_INFO = plsc.get_sparse_core_info()
_NC = _SC_INFO.num_cores
_NS = _SC_INFO.num_subcores
_NW = _NC * _NS
_ROWS_PER_W = B // _NW
_NVH = HIW // 16
_SU = 6

_sc_mesh = plsc.VectorSubcoreMesh(core_axis_name="c", subcore_axis_name="s")


@functools.partial(
    pl.kernel,
    mesh=_sc_mesh,
    out_type=jax.ShapeDtypeStruct((B, HIW), jnp.uint32),
    scratch_types=[pltpu.VMEM((HIW,), jnp.uint32)],
)
def _sc_bits(out_hbm, row_v):
    wid = lax.axis_index("s") * _NC + lax.axis_index("c")
    iota16 = lax.iota(jnp.uint32, 16)

    def do_row(j, _):
        r = wid * _ROWS_PER_W + j
        rbase = jnp.uint32(r) * jnp.uint32(N) + jnp.uint32(CLO)

        def step(i, _):
            b = i * _SU * 16
            for k in range(_SU):
                off = b + k * 16
                idx = rbase + jnp.uint32(off) + iota16
                row_v[pl.ds(off, 16)] = _threefry_bits(idx)
            return 0

        lax.fori_loop(0, _NVH // _SU, step, 0)
        pltpu.sync_copy(row_v, out_hbm.at[r])
        return 0

    lax.fori_loop(0, _ROWS_PER_W, do_row, 0)


@jax.jit
def kernel(x):
    bits = _sc_bits()
    probs, ehi, wmin, widx = pl.pallas_call(
        _body_a,
        grid=(GRID,),
        in_specs=[pl.BlockSpec((BR, N), lambda i: (i, 0))],
        out_specs=[
            pl.BlockSpec((BR, N), lambda i: (i, 0)),
            pl.BlockSpec((BR, HIW), lambda i: (i, 0)),
            pl.BlockSpec((1, BR, W), lambda i: (i, 0, 0)),
            pl.BlockSpec((1, BR, W), lambda i: (i, 0, 0)),
        ],
        out_shape=[
            jax.ShapeDtypeStruct((B, N), jnp.float32),
            jax.ShapeDtypeStruct((B, HIW), jnp.float32),
            jax.ShapeDtypeStruct((GRID, BR, W), jnp.float32),
            jax.ShapeDtypeStruct((GRID, BR, W), jnp.int32),
        ],
    )(x)
    y3 = pl.pallas_call(
        _body_b,
        grid=(GRID,),
        in_specs=[
            pl.BlockSpec((BR, HIW), lambda i: (i, 0)),
            pl.BlockSpec((BR, HIW), lambda i: (i, 0)),
            pl.BlockSpec((1, BR, W), lambda i: (i, 0, 0)),
            pl.BlockSpec((1, BR, W), lambda i: (i, 0, 0)),
        ],
        out_specs=pl.BlockSpec((1, 1, BR), lambda i: (i, 0, 0)),
        out_shape=jax.ShapeDtypeStruct((GRID, 1, BR), jnp.int32),
    )(bits, ehi, wmin, widx)
    return (y3.reshape(B), probs)
